# baseline (speedup 1.0000x reference)
import functools

import jax
import jax.numpy as jnp
from jax import lax
from jax.experimental import pallas as pl
from jax.experimental.pallas import tpu as pltpu
from jax.experimental.pallas import tpu_sc as plsc

VOCAB = 1000000
EMBED = 64
HIDDEN = 1000
NUM_CLASS = 100
BATCH = 4096
SEQ = 200

NC = 2
NS = 16
NW = NC * NS
LANES = 16
NV = EMBED // LANES

B_PER_W = BATCH // NW
CHUNK_A = 104
CHUNK_B = 96


def _embed_mean(input_ids, table):
  mesh = plsc.VectorSubcoreMesh(core_axis_name="c", subcore_axis_name="s")

  @functools.partial(
      pl.kernel,
      mesh=mesh,
      compiler_params=pltpu.CompilerParams(use_tc_tiling_on_sc=False),
      out_type=jax.ShapeDtypeStruct((BATCH, EMBED), jnp.float32),
      scratch_types=[
          pltpu.VMEM((B_PER_W, CHUNK_A), jnp.int32),
          pltpu.VMEM((B_PER_W, CHUNK_B), jnp.int32),
          pltpu.VMEM((2, CHUNK_A, EMBED), jnp.float32),
          pltpu.VMEM((2, CHUNK_B, EMBED), jnp.float32),
          pltpu.VMEM((B_PER_W, EMBED), jnp.float32),
          pltpu.SemaphoreType.DMA,
          pltpu.SemaphoreType.DMA,
      ],
  )
  def k(ids_hbm, tab_hbm, out_hbm, hA, hB, rowsA, rowsB, acc_v, sem0, sem1):
    wid = lax.axis_index("s") * NC + lax.axis_index("c")
    row_base = wid * B_PER_W
    rsl = pl.ds(row_base, B_PER_W)
    sems = (sem0, sem1)

    pltpu.sync_copy(ids_hbm.at[rsl, pl.ds(0, CHUNK_A)], hA)
    pltpu.sync_copy(ids_hbm.at[rsl, pl.ds(CHUNK_A, CHUNK_B)], hB)

    def fire(b, buf):
      pltpu.async_copy(tab_hbm.at[hA.at[b]], rowsA.at[buf], sems[buf])
      pltpu.async_copy(tab_hbm.at[hB.at[b]], rowsB.at[buf], sems[buf])

    def drain(buf):
      pltpu.make_async_copy(tab_hbm.at[hA.at[0]], rowsA.at[buf],
                            sems[buf]).wait()
      pltpu.make_async_copy(tab_hbm.at[hB.at[0]], rowsB.at[buf],
                            sems[buf]).wait()

    zero = jnp.zeros((LANES,), jnp.float32)

    def consume(b, buf):
      def make_sum(rows_ref):
        def sum_body(j, carry):
          acc = list(carry)
          for v in range(NV):
            acc[v] += rows_ref[j, pl.ds(v * LANES, LANES)]
          return tuple(acc)
        return sum_body

      sums = lax.fori_loop(0, CHUNK_A, make_sum(rowsA.at[buf]), (zero,) * NV)
      sums = lax.fori_loop(0, CHUNK_B, make_sum(rowsB.at[buf]), sums)
      inv = jnp.float32(1.0 / SEQ)
      for v in range(NV):
        acc_v[b, pl.ds(v * LANES, LANES)] = sums[v] * inv

    fire(0, 0)

    def pair_body(i, _):
      b0 = 2 * i
      fire(b0 + 1, 1)
      drain(0)
      consume(b0, 0)

      @pl.when(i < B_PER_W // 2 - 1)
      def _():
        fire(b0 + 2, 0)

      drain(1)
      consume(b0 + 1, 1)
      return 0

    lax.fori_loop(0, B_PER_W // 2, pair_body, 0)

    pltpu.sync_copy(acc_v, out_hbm.at[rsl])

  return k(input_ids, table)


TB = 47616
NBLK = (VOCAB + TB - 1) // TB
VOCAB_PAD = NBLK * TB


def _pack_body(tin_ref, o_ref):
  xt = tin_ref[...].T
  o_ref[:, 0:EMBED] = xt[0:TB // 2, :]
  o_ref[:, EMBED:2 * EMBED] = xt[TB // 2:TB, :]


def _pack_table(tableT):
  return pl.pallas_call(
      _pack_body,
      grid=(NBLK,),
      in_specs=[pl.BlockSpec((EMBED, TB), lambda i: (0, i))],
      out_specs=pl.BlockSpec((TB // 2, 2 * EMBED), lambda i: (i, 0)),
      out_shape=jax.ShapeDtypeStruct((VOCAB_PAD // 2, 2 * EMBED),
                                     jnp.float32),
  )(tableT)


def _mlp_body(x_ref, w1_ref, b1_ref, w2_ref, b2_ref, o_ref):
  x = x_ref[...]
  h = lax.dot_general(x, w1_ref[...], (((1,), (1,)), ((), ())),
                      preferred_element_type=jnp.float32)
  h = jnp.maximum(h + b1_ref[...], 0.0)
  o = lax.dot_general(h, w2_ref[...], (((1,), (1,)), ((), ())),
                      preferred_element_type=jnp.float32)
  o_ref[...] = o + b2_ref[...]


def _mlp(embedded, W1, b1, W2, b2):
  BB = 512
  grid = (BATCH // BB,)
  return pl.pallas_call(
      _mlp_body,
      grid=grid,
      in_specs=[
          pl.BlockSpec((BB, EMBED), lambda i: (i, 0)),
          pl.BlockSpec((HIDDEN, EMBED), lambda i: (0, 0)),
          pl.BlockSpec((1, HIDDEN), lambda i: (0, 0)),
          pl.BlockSpec((NUM_CLASS, HIDDEN), lambda i: (0, 0)),
          pl.BlockSpec((1, NUM_CLASS), lambda i: (0, 0)),
      ],
      out_specs=pl.BlockSpec((BB, NUM_CLASS), lambda i: (i, 0)),
      out_shape=jax.ShapeDtypeStruct((BATCH, NUM_CLASS), jnp.float32),
  )(embedded, W1, b1.reshape(1, HIDDEN), W2, b2.reshape(1, NUM_CLASS))


def kernel(input_ids, table, W1, b1, W2, b2):
  tab_lin = _pack_table(table.T).reshape(VOCAB_PAD, EMBED)
  u = input_ids % TB
  ids2 = (input_ids - u) + jnp.where(
      u < TB // 2, 2 * u, 2 * (u - TB // 2) + 1)
  embedded = _embed_mean(ids2, tab_lin)
  return _mlp(embedded, W1, b1, W2, b2)


# speedup vs baseline: 1.0200x; 1.0200x over previous
import functools

import jax
import jax.numpy as jnp
from jax import lax
from jax.experimental import pallas as pl
from jax.experimental.pallas import tpu as pltpu
from jax.experimental.pallas import tpu_sc as plsc

VOCAB = 1000000
EMBED = 64
HIDDEN = 1000
NUM_CLASS = 100
BATCH = 4096
SEQ = 200

NC = 2
NS = 16
NW = NC * NS
LANES = 16
NV = EMBED // LANES

B_PER_W = BATCH // NW
CHUNK_A = 104
CHUNK_B = 96


def _embed_mean(input_ids, table):
  mesh = plsc.VectorSubcoreMesh(core_axis_name="c", subcore_axis_name="s")

  @functools.partial(
      pl.kernel,
      mesh=mesh,
      compiler_params=pltpu.CompilerParams(use_tc_tiling_on_sc=False),
      out_type=jax.ShapeDtypeStruct((BATCH, EMBED), jnp.float32),
      scratch_types=[
          pltpu.VMEM((B_PER_W, CHUNK_A), jnp.int32),
          pltpu.VMEM((B_PER_W, CHUNK_B), jnp.int32),
          pltpu.VMEM((2, CHUNK_A, EMBED), jnp.float32),
          pltpu.VMEM((2, CHUNK_B, EMBED), jnp.float32),
          pltpu.VMEM((B_PER_W, EMBED), jnp.float32),
          pltpu.SemaphoreType.DMA,
          pltpu.SemaphoreType.DMA,
      ],
  )
  def k(ids_hbm, tab_hbm, out_hbm, hA, hB, rowsA, rowsB, acc_v, sem0, sem1):
    wid = lax.axis_index("s") * NC + lax.axis_index("c")
    row_base = wid * B_PER_W
    rsl = pl.ds(row_base, B_PER_W)
    sems = (sem0, sem1)

    pltpu.sync_copy(ids_hbm.at[rsl, pl.ds(0, CHUNK_A)], hA)
    pltpu.sync_copy(ids_hbm.at[rsl, pl.ds(CHUNK_A, CHUNK_B)], hB)

    def fire(b, buf):
      pltpu.async_copy(tab_hbm.at[hA.at[b]], rowsA.at[buf], sems[buf])
      pltpu.async_copy(tab_hbm.at[hB.at[b]], rowsB.at[buf], sems[buf])

    def drain(buf):
      pltpu.make_async_copy(tab_hbm.at[hA.at[0]], rowsA.at[buf],
                            sems[buf]).wait()
      pltpu.make_async_copy(tab_hbm.at[hB.at[0]], rowsB.at[buf],
                            sems[buf]).wait()

    zero = jnp.zeros((LANES,), jnp.float32)

    def consume(b, buf):
      def make_sum(rows_ref):
        def sum_body(j, carry):
          acc = list(carry)
          for v in range(NV):
            acc[v] += rows_ref[j, pl.ds(v * LANES, LANES)]
          return tuple(acc)
        return sum_body

      sums = lax.fori_loop(0, CHUNK_A, make_sum(rowsA.at[buf]), (zero,) * NV)
      sums = lax.fori_loop(0, CHUNK_B, make_sum(rowsB.at[buf]), sums)
      inv = jnp.float32(1.0 / SEQ)
      for v in range(NV):
        acc_v[b, pl.ds(v * LANES, LANES)] = sums[v] * inv

    fire(0, 0)

    def pair_body(i, _):
      b0 = 2 * i
      fire(b0 + 1, 1)
      drain(0)
      consume(b0, 0)

      @pl.when(i < B_PER_W // 2 - 1)
      def _():
        fire(b0 + 2, 0)

      drain(1)
      consume(b0 + 1, 1)
      return 0

    lax.fori_loop(0, B_PER_W // 2, pair_body, 0)

    pltpu.sync_copy(acc_v, out_hbm.at[rsl])

  return k(input_ids, table)


TB = 31744
NBLK = (VOCAB + TB - 1) // TB
VOCAB_PAD = NBLK * TB


def _pack_body(tin_ref, o_ref):
  xt = tin_ref[...].T
  o_ref[:, 0:EMBED] = xt[0:TB // 2, :]
  o_ref[:, EMBED:2 * EMBED] = xt[TB // 2:TB, :]


def _pack_table(tableT):
  return pl.pallas_call(
      _pack_body,
      grid=(NBLK,),
      in_specs=[pl.BlockSpec((EMBED, TB), lambda i: (0, i))],
      out_specs=pl.BlockSpec((TB // 2, 2 * EMBED), lambda i: (i, 0)),
      out_shape=jax.ShapeDtypeStruct((VOCAB_PAD // 2, 2 * EMBED),
                                     jnp.float32),
  )(tableT)


def _mlp_body(x_ref, w1_ref, b1_ref, w2_ref, b2_ref, o_ref):
  x = x_ref[...]
  h = lax.dot_general(x, w1_ref[...], (((1,), (1,)), ((), ())),
                      preferred_element_type=jnp.float32)
  h = jnp.maximum(h + b1_ref[...], 0.0)
  o = lax.dot_general(h, w2_ref[...], (((1,), (1,)), ((), ())),
                      preferred_element_type=jnp.float32)
  o_ref[...] = o + b2_ref[...]


def _mlp(embedded, W1, b1, W2, b2):
  BB = 512
  grid = (BATCH // BB,)
  return pl.pallas_call(
      _mlp_body,
      grid=grid,
      in_specs=[
          pl.BlockSpec((BB, EMBED), lambda i: (i, 0)),
          pl.BlockSpec((HIDDEN, EMBED), lambda i: (0, 0)),
          pl.BlockSpec((1, HIDDEN), lambda i: (0, 0)),
          pl.BlockSpec((NUM_CLASS, HIDDEN), lambda i: (0, 0)),
          pl.BlockSpec((1, NUM_CLASS), lambda i: (0, 0)),
      ],
      out_specs=pl.BlockSpec((BB, NUM_CLASS), lambda i: (i, 0)),
      out_shape=jax.ShapeDtypeStruct((BATCH, NUM_CLASS), jnp.float32),
  )(embedded, W1, b1.reshape(1, HIDDEN), W2, b2.reshape(1, NUM_CLASS))


def kernel(input_ids, table, W1, b1, W2, b2):
  tab_lin = _pack_table(table.T).reshape(VOCAB_PAD, EMBED)
  u = input_ids % TB
  ids2 = (input_ids - u) + jnp.where(
      u < TB // 2, 2 * u, 2 * (u - TB // 2) + 1)
  embedded = _embed_mean(ids2, tab_lin)
  return _mlp(embedded, W1, b1, W2, b2)
